# baseline (speedup 1.0000x reference)
import jax
import jax.numpy as jnp
from jax.experimental import pallas as pl


def _net_kernel(x_ref, a_ref, w1_ref, b1_ref, w2_ref, b2_ref, o_ref):
    n = a_ref.shape[0]
    h_dim = o_ref.shape[1]
    a = a_ref[...]
    row = jax.lax.broadcasted_iota(jnp.int32, (n, n), 0)
    col = jax.lax.broadcasted_iota(jnp.int32, (n, n), 1)
    af = jnp.where((row == col) | a, 1.0, 0.0).astype(jnp.bfloat16)

    tdot = lambda m, y: jax.lax.dot_general(
        m, y, (((0,), (0,)), ((), ())), preferred_element_type=jnp.float32
    )

    ones = jnp.ones((h_dim, h_dim), dtype=jnp.float32)

    def l2norm(x):
        nsq = jax.lax.dot_general(
            x * x, ones, (((1,), (0,)), ((), ())),
            preferred_element_type=jnp.float32,
        )
        return x * jax.lax.rsqrt(jnp.maximum(nsq, 1e-24))

    y1 = jnp.dot(x_ref[...], w1_ref[...], preferred_element_type=jnp.float32)
    h = l2norm(tdot(af, y1.astype(jnp.bfloat16)) + b1_ref[...])
    h = jnp.maximum(h, 0.0)

    y2 = jnp.dot(h, w2_ref[...], preferred_element_type=jnp.float32)
    o = l2norm(tdot(af, y2.astype(jnp.bfloat16)) + b2_ref[...])
    o_ref[...] = l2norm(o)


def kernel(X, A, W1, b1, W2, b2):
    n = X.shape[0]
    h = W1.shape[1]
    return pl.pallas_call(
        _net_kernel,
        out_shape=jax.ShapeDtypeStruct((n, h), jnp.float32),
    )(X, A, W1, b1.reshape(1, h), W2, b2.reshape(1, h))
